# baseline (speedup 1.0000x reference)
import functools

import jax
import jax.numpy as jnp
from jax import lax
from jax.experimental import pallas as pl
from jax.experimental.pallas import tpu as pltpu
from jax.experimental.pallas import tpu_sc as plsc

N = 50000
E = 800000
B = 128
NCHUNK = 400
EPT = NCHUNK * B
E_PAD = EPT * 16
NBUF = 4
N_PAD = 50176
RPT = N_PAD // 16
RCH = 112


def _seg_feat_kernel(F):
    mesh = plsc.VectorSubcoreMesh(core_axis_name="c", subcore_axis_name="s")

    scratch = (
        [pltpu.VMEM((B,), jnp.int32) for _ in range(NBUF)]
        + [pltpu.VMEM((B,), jnp.float32) for _ in range(NBUF)]
        + [pltpu.VMEM((B,), jnp.int32) for _ in range(NBUF)]
        + [pltpu.VMEM((B, F), jnp.float32) for _ in range(NBUF)]
        + [
            pltpu.VMEM((RCH, F), jnp.float32),
            pltpu.VMEM((RCH, F), jnp.float32),
            pltpu.VMEM_SHARED((N_PAD, F), jnp.float32),
        ]
        + [pltpu.SemaphoreType.DMA for _ in range(4 * NBUF)]
    )

    @functools.partial(
        pl.kernel,
        mesh=mesh,
        compiler_params=pltpu.CompilerParams(use_tc_tiling_on_sc=False),
        out_type=jax.ShapeDtypeStruct((N_PAD, 2, F), jnp.float32),
        scratch_types=scratch,
    )
    def k(*refs):
        xa_hbm, xb_hbm, src_hbm, dst_hbm, w_hbm, out_hbm = refs[:6]
        rest = refs[6:]
        src_v = rest[0:NBUF]
        w_v = rest[NBUF:2 * NBUF]
        dst_v = rest[2 * NBUF:3 * NBUF]
        rows_v = rest[3 * NBUF:4 * NBUF]
        zbuf_v, stage_v, spacc = rest[4 * NBUF:4 * NBUF + 3]
        sems = rest[4 * NBUF + 3:]
        se = sems[0:NBUF]
        sd = sems[NBUF:2 * NBUF]
        sg = sems[2 * NBUF:3 * NBUF]
        ss = sems[3 * NBUF:4 * NBUF]

        c = lax.axis_index("c")
        s = lax.axis_index("s")
        ebase = s * EPT
        zeros16 = jnp.zeros((16,), jnp.float32)

        def zero_stage(r, _):
            for j in range(F // 16):
                zbuf_v[r, pl.ds(j * 16, 16)] = zeros16
            return 0

        lax.fori_loop(0, RCH, zero_stage, 0)

        def issue_ew(ch, b):
            base = ebase + ch * B
            pltpu.async_copy(src_hbm.at[pl.ds(base, B)], src_v[b], se[b])
            pltpu.async_copy(w_hbm.at[pl.ds(base, B)], w_v[b], se[b])

        def wait_ew(b):
            pltpu.make_async_copy(src_hbm.at[pl.ds(0, B)], src_v[b],
                                  se[b]).wait()
            pltpu.make_async_copy(w_hbm.at[pl.ds(0, B)], w_v[b],
                                  se[b]).wait()

        def issue_dst(ch, b):
            base = ebase + ch * B
            pltpu.async_copy(dst_hbm.at[pl.ds(base, B)], dst_v[b], sd[b])

        def wait_dst(b):
            pltpu.make_async_copy(dst_hbm.at[pl.ds(0, B)], dst_v[b],
                                  sd[b]).wait()

        def scale(b):
            def scale_grp(g, _):
                w16 = w_v[b][pl.ds(g * 16, 16)]
                for lane in range(16):
                    e = g * 16 + lane
                    wv = w16[lane]
                    for j in range(F // 16):
                        sl = pl.ds(j * 16, 16)
                        rows_v[b][e, sl] = rows_v[b][e, sl] * wv
                return 0

            lax.fori_loop(0, B // 16, scale_grp, 0)

        def gather(b):
            def ga(x_hbm):
                pltpu.async_copy(x_hbm.at[src_v[b]], rows_v[b], sg[b])

            pl.when(c == 0)(lambda: ga(xa_hbm))
            pl.when(c == 1)(lambda: ga(xb_hbm))

        def wait_gather(b):
            pltpu.make_async_copy(xa_hbm.at[src_v[b]], rows_v[b],
                                  sg[b]).wait()

        def zero_acc(kk, _):
            pltpu.sync_copy(zbuf_v,
                            spacc.at[pl.ds(s * RPT + kk * RCH, RCH)])
            return 0

        lax.fori_loop(0, RPT // RCH, zero_acc, 0)
        plsc.subcore_barrier()

        for b in range(NBUF):
            issue_ew(b, b)
        issue_dst(0, 0)
        wait_ew(0)
        gather(0)

        def outer(kk, _):
            for u in range(NBUF):
                bcur = u
                bnext = (u + 1) % NBUF
                ch = NBUF * kk + u

                def w1():
                    pltpu.make_async_copy(
                        rows_v[bnext],
                        spacc.at[dst_v[bnext]], ss[bnext]).wait()

                if u == NBUF - 1:
                    w1()
                else:
                    pl.when(kk > 0)(w1)

                def advance():
                    issue_dst(ch + 1, bnext)
                    wait_ew(bnext)
                    gather(bnext)

                if u == NBUF - 1:
                    pl.when(kk < (NCHUNK // NBUF) - 1)(advance)
                else:
                    advance()

                wait_gather(bcur)
                scale(bcur)

                def reload():
                    issue_ew(ch + NBUF, bcur)

                pl.when(kk < (NCHUNK // NBUF) - 1)(reload)

                wait_dst(bcur)
                pltpu.async_copy(rows_v[bcur], spacc.at[dst_v[bcur]],
                                 ss[bcur], add=True)
            return 0

        lax.fori_loop(0, NCHUNK // NBUF, outer, 0)

        for i in range(NBUF - 1):
            b = (NCHUNK - (NBUF - 1) + i) % NBUF
            pltpu.make_async_copy(rows_v[b], spacc.at[dst_v[b]],
                                  ss[b]).wait()
        plsc.subcore_barrier()

        def writeout(kk, _):
            r0 = s * RPT + kk * RCH
            pltpu.sync_copy(spacc.at[pl.ds(r0, RCH)], stage_v)
            pltpu.sync_copy(stage_v, out_hbm.at[pl.ds(r0, RCH), c])
            return 0

        lax.fori_loop(0, RPT // RCH, writeout, 0)

    return k


_segf16 = _seg_feat_kernel(16)
_segf32 = _seg_feat_kernel(32)


def _dense_block(agg, x, Wr, br, Wt, skip):
    d = x.shape[1]
    ROWS = 400
    grid = (N // ROWS,)

    def body(a_ref, x_ref, br_ref, wr_ref, wt_ref, o_ref):
        acc = br_ref[...] + jnp.dot(x_ref[...], wt_ref[...],
                                    preferred_element_type=jnp.float32)
        acc = acc + jnp.dot(a_ref[...], wr_ref[...],
                            preferred_element_type=jnp.float32)
        acc = jnp.maximum(acc, 0.0)
        if skip:
            acc = jnp.maximum(acc + x_ref[...], 0.0)
        o_ref[...] = acc

    row_spec = lambda w: pl.BlockSpec((ROWS, w), lambda i: (i, 0))
    full_spec = lambda a, b: pl.BlockSpec((a, b), lambda i: (0, 0))
    return pl.pallas_call(
        body,
        grid=grid,
        in_specs=[row_spec(d), row_spec(d), full_spec(1, 64),
                  full_spec(d, 64), full_spec(d, 64)],
        out_specs=row_spec(64),
        out_shape=jax.ShapeDtypeStruct((N, 64), jnp.float32),
    )(agg, x, br.reshape(1, 64), Wr, Wt)


def kernel(z, edge_index, edge_weight, batch,
           W_rel0, b_rel0, W_root0,
           W_rel1, b_rel1, W_root1,
           W_rel2, b_rel2, W_root2,
           W_rel3, b_rel3, W_root3):
    pad = E_PAD - E
    src = jnp.concatenate([edge_index[0], jnp.zeros((pad,), jnp.int32)])
    dst = jnp.concatenate([edge_index[1], jnp.zeros((pad,), jnp.int32)])
    w = jnp.concatenate([edge_weight, jnp.zeros((pad,), jnp.float32)])

    def block(x, Wr, br, Wt, skip):
        d = x.shape[1]
        fh = d // 2
        seg = _segf16 if d == 32 else _segf32
        p = seg(x[:, :fh], x[:, fh:], src, dst, w)
        agg = p.reshape(N_PAD, d)
        return _dense_block(agg, x, Wr, br, Wt, skip)

    h = block(z, W_rel0, b_rel0, W_root0, False)
    h = block(h, W_rel1, b_rel1, W_root1, True)
    h = block(h, W_rel2, b_rel2, W_root2, True)
    h = block(h, W_rel3, b_rel3, W_root3, False)
    return h
